# baseline (speedup 1.0000x reference)
import dataclasses
import functools

import jax
import jax.numpy as jnp
from jax import lax
from jax.experimental import pallas as pl
from jax.experimental.pallas import tpu as pltpu
from jax.experimental.pallas import tpu_sc as plsc

N = 10000
D = 128
S = 5
E = 120000

NSUB = 16
LANES = 16
WSZ = 128
HALF = 64
EDGES = S * E
SLOTS = 293
LAST = SLOTS - 1
RCH = 80
NCH = N // RCH


def _matmul_table(x, wt):
    m = x.shape[0]
    mb = 2000
    nb = m // mb

    def body(wt_ref, x_ref, z_ref, wacc_ref):
        i = pl.program_id(1)

        @pl.when(i == 0)
        def _():
            wacc_ref[...] = jnp.zeros_like(wacc_ref)

        wacc_ref[...] += wt_ref[0]
        y = jnp.dot(x_ref[...], wacc_ref[...].astype(jnp.bfloat16),
                    preferred_element_type=jnp.float32)
        z_ref[0] = y

    return pl.pallas_call(
        body,
        grid=(nb, S),
        in_specs=[
            pl.BlockSpec((1, D, D), lambda j, i: (i, 0, 0)),
            pl.BlockSpec((mb, D), lambda j, i: (j, 0)),
        ],
        out_specs=pl.BlockSpec((1, mb, D), lambda j, i: (i, j, 0)),
        out_shape=jax.ShapeDtypeStruct((S, m, D), jnp.float32),
        scratch_shapes=[pltpu.VMEM((D, D), jnp.float32)],
    )(wt, x)


def _sc_aggregate(z, gidx_u, rows_u, vals_u, gidx_i, rows_i, vals_i):
    mesh = plsc.VectorSubcoreMesh(core_axis_name="c", subcore_axis_name="s")
    cp = pltpu.CompilerParams()
    if "needs_layout_passes" in pltpu.CompilerParams.__dataclass_fields__:
        cp = dataclasses.replace(cp, needs_layout_passes=False)

    @functools.partial(
        pl.kernel,
        out_type=[jax.ShapeDtypeStruct((N, D), jnp.float32),
                  jax.ShapeDtypeStruct((N, D), jnp.float32)],
        mesh=mesh,
        compiler_params=cp,
        scratch_types=[
            pltpu.VMEM((WSZ,), jnp.int32),
            pltpu.VMEM((WSZ,), jnp.int32),
            pltpu.VMEM((WSZ,), jnp.int32),
            pltpu.VMEM((WSZ,), jnp.int32),
            pltpu.VMEM((WSZ,), jnp.float32),
            pltpu.VMEM((WSZ,), jnp.float32),
            pltpu.VMEM((WSZ, D), jnp.float32),
            pltpu.VMEM((WSZ, D), jnp.float32),
            pltpu.VMEM((WSZ,), jnp.int32),
            pltpu.VMEM((WSZ,), jnp.int32),
            pltpu.VMEM_SHARED((N, D), jnp.float32),
            pltpu.SemaphoreType.DMA,
            pltpu.SemaphoreType.DMA,
            pltpu.SemaphoreType.DMA,
            pltpu.SemaphoreType.DMA,
            pltpu.SemaphoreType.DMA,
            pltpu.SemaphoreType.DMA,
        ],
    )
    def agg(z_hbm, gu_hbm, ru_hbm, vu_hbm, gi_hbm, ri_hbm, vi_hbm,
            outu_hbm, outi_hbm,
            idx0, idx1, rows0, rows1, val0, val1, gbuf0, gbuf1,
            srows0, srows1, acc_sh, esem0, esem1, gsem0, gsem1,
            ssem0, ssem1):
        c = lax.axis_index("c")
        s = lax.axis_index("s")
        idxb, rowsb, valb = [idx0, idx1], [rows0, rows1], [val0, val1]
        gb = [gbuf0, gbuf1]
        srows = [srows0, srows1]
        esem, gsem = [esem0, esem1], [gsem0, gsem1]
        ssem = [ssem0, ssem1]
        zero16 = jnp.zeros((LANES,), jnp.float32)

        def run_side(e_hbm, r_hbm, v_hbm, out_hbm):
            def e_descs(w, b, sz):
                base = (s + NSUB * w) * WSZ
                dst = ((idxb[b], rowsb[b], valb[b]) if sz == WSZ else
                       (idxb[b].at[pl.ds(0, sz)], rowsb[b].at[pl.ds(0, sz)],
                        valb[b].at[pl.ds(0, sz)]))
                return [
                    pltpu.make_async_copy(e_hbm.at[pl.ds(base, sz)], dst[0], esem[b]),
                    pltpu.make_async_copy(r_hbm.at[pl.ds(base, sz)], dst[1], esem[b]),
                    pltpu.make_async_copy(v_hbm.at[pl.ds(base, sz)], dst[2], esem[b]),
                ]

            def e_start(w, b):
                part = jnp.logical_and(w == LAST, s == NSUB - 1)

                @pl.when(part)
                def _():
                    for cpy in e_descs(w, b, HALF):
                        cpy.start()

                @pl.when(jnp.logical_not(part))
                def _():
                    for cpy in e_descs(w, b, WSZ):
                        cpy.start()

            def e_wait(w, b):
                part = jnp.logical_and(w == LAST, s == NSUB - 1)

                @pl.when(part)
                def _():
                    for cpy in e_descs(w, b, HALF):
                        cpy.wait()
                    for k in range(HALF // LANES):
                        valb[b][pl.ds(HALF + k * LANES, LANES)] = zero16

                @pl.when(jnp.logical_not(part))
                def _():
                    for cpy in e_descs(w, b, WSZ):
                        cpy.wait()

            def gcopy(b):
                return pltpu.make_async_copy(z_hbm.at[idxb[b]], gb[b], gsem[b])

            def scopy(b):
                return pltpu.make_async_copy(gb[b], acc_sh.at[srows[b]], ssem[b])

            e_start(0, 0)
            e_start(1, 1)

            @pl.loop(0, RCH)
            def _(r):
                for k in range(D // LANES):
                    gbuf1[r, pl.ds(k * LANES, LANES)] = zero16

            @pl.loop(0, 8)
            def _(k):
                ch = s + NSUB * k

                @pl.when(ch < NCH)
                def _():
                    pltpu.sync_copy(gbuf1.at[pl.ds(0, RCH)],
                                    acc_sh.at[pl.ds(ch * RCH, RCH)])

            e_wait(0, 0)
            gcopy(0).start()

            plsc.subcore_barrier()

            def half_iter(w, p):
                q = 1 - p

                @pl.when(w >= 1)
                def _():
                    scopy(q).wait()

                @pl.when(w + 1 <= LAST)
                def _():
                    e_wait(w + 1, q)
                    gcopy(q).start()

                gcopy(p).wait()

                for k in range(WSZ // LANES):
                    sl = pl.ds(k * LANES, LANES)
                    srows[p][sl] = rowsb[p][sl]

                @pl.loop(0, WSZ, unroll=4)
                def _(e):
                    vv = plsc.load_gather(
                        valb[p], [jnp.full((LANES,), e, jnp.int32)])
                    for k in range(D // LANES):
                        sl = pl.ds(k * LANES, LANES)
                        gb[p][e, sl] = gb[p][e, sl] * vv

                pltpu.async_copy(gb[p], acc_sh.at[srows[p]], ssem[p], add=True)

                @pl.when(w + 2 <= LAST)
                def _():
                    e_start(w + 2, p)

            @pl.loop(0, SLOTS + 1, step=2)
            def _(w0):
                for p in (0, 1):
                    w = w0 + p

                    @pl.when(w <= LAST)
                    def _():
                        half_iter(w, p)

            scopy(LAST % 2).wait()
            plsc.subcore_barrier()

            @pl.loop(0, 8)
            def _(k):
                ch = s + NSUB * k

                @pl.when(ch < NCH)
                def _():
                    pltpu.sync_copy(acc_sh.at[pl.ds(ch * RCH, RCH)],
                                    gbuf0.at[pl.ds(0, RCH)])

                    @pl.loop(0, RCH, unroll=4)
                    def _(r):
                        for kk in range(D // LANES):
                            sl = pl.ds(kk * LANES, LANES)
                            gbuf0[r, sl] = jnp.maximum(gbuf0[r, sl], 0.0)

                    pltpu.sync_copy(gbuf0.at[pl.ds(0, RCH)],
                                    out_hbm.at[pl.ds(ch * RCH, RCH)])

        @pl.when(c == 0)
        def _():
            run_side(gu_hbm, ru_hbm, vu_hbm, outu_hbm)

        @pl.when(c == 1)
        def _():
            run_side(gi_hbm, ri_hbm, vi_hbm, outi_hbm)

    return agg(z, gidx_u, rows_u, vals_u, gidx_i, rows_i, vals_i)


def kernel(user_sup_idx, user_sup_val, item_sup_idx, item_sup_val,
           user_inputs, item_inputs, W):
    x = jnp.concatenate([user_inputs, item_inputs],
                        axis=0).astype(jnp.bfloat16)
    wt = jnp.transpose(W, (2, 0, 1))
    z = _matmul_table(x, wt).reshape(S * 2 * N, D)

    sup_off = (jnp.arange(S, dtype=jnp.int32) * (2 * N))[:, None]
    gidx_u = (user_sup_idx[:, 1, :] + sup_off + N).reshape(-1)
    gidx_i = (item_sup_idx[:, 1, :] + sup_off).reshape(-1)
    rows_u = user_sup_idx[:, 0, :].reshape(-1)
    rows_i = item_sup_idx[:, 0, :].reshape(-1)
    vals_u = user_sup_val.reshape(-1)
    vals_i = item_sup_val.reshape(-1)

    out_u, out_i = _sc_aggregate(z, gidx_u, rows_u, vals_u,
                                 gidx_i, rows_i, vals_i)
    return (out_u, out_i)


# speedup vs baseline: 1.0349x; 1.0349x over previous
import dataclasses
import functools

import jax
import jax.numpy as jnp
from jax import lax
from jax.experimental import pallas as pl
from jax.experimental.pallas import tpu as pltpu
from jax.experimental.pallas import tpu_sc as plsc

N = 10000
D = 128
S = 5
E = 120000

NSUB = 16
LANES = 16
WSZ = 96
NWS = E // WSZ
NW = S * NWS
VSLOTS = 392
RCH = 80
NCH = N // RCH


def _matmul_table(x, wt):
    m = x.shape[0]
    mb = 4000
    nb = m // mb

    def body(wt_ref, x_ref, z_ref, wacc_ref):
        i = pl.program_id(1)

        @pl.when(i == 0)
        def _():
            wacc_ref[...] = jnp.zeros_like(wacc_ref)

        wacc_ref[...] += wt_ref[0]
        z_ref[0] = jnp.dot(x_ref[...], wacc_ref[...].astype(jnp.bfloat16),
                           preferred_element_type=jnp.float32)

    return pl.pallas_call(
        body,
        grid=(nb, S),
        in_specs=[
            pl.BlockSpec((1, D, D), lambda j, i: (i, 0, 0)),
            pl.BlockSpec((mb, D), lambda j, i: (j, 0)),
        ],
        out_specs=pl.BlockSpec((1, mb, D), lambda j, i: (i, j, 0)),
        out_shape=jax.ShapeDtypeStruct((S, m, D), jnp.float32),
        scratch_shapes=[pltpu.VMEM((D, D), jnp.float32)],
    )(wt, x)


def _sc_aggregate(z, su_u, sv_u, su_i, sv_i):
    mesh = plsc.VectorSubcoreMesh(core_axis_name="c", subcore_axis_name="s")
    cp = pltpu.CompilerParams()
    if "needs_layout_passes" in pltpu.CompilerParams.__dataclass_fields__:
        cp = dataclasses.replace(cp, needs_layout_passes=False)

    @functools.partial(
        pl.kernel,
        out_type=[jax.ShapeDtypeStruct((N, D), jnp.float32),
                  jax.ShapeDtypeStruct((N, D), jnp.float32)],
        mesh=mesh,
        compiler_params=cp,
        scratch_types=[
            pltpu.VMEM((WSZ,), jnp.int32),
            pltpu.VMEM((WSZ,), jnp.int32),
            pltpu.VMEM((WSZ,), jnp.int32),
            pltpu.VMEM((WSZ,), jnp.int32),
            pltpu.VMEM((WSZ,), jnp.float32),
            pltpu.VMEM((WSZ,), jnp.float32),
            pltpu.VMEM((WSZ, D), jnp.float32),
            pltpu.VMEM((WSZ, D), jnp.float32),
            pltpu.VMEM((RCH, D), jnp.float32),
            pltpu.VMEM((WSZ,), jnp.int32),
            pltpu.VMEM((WSZ,), jnp.int32),
            pltpu.VMEM_SHARED((N, D), jnp.float32),
            pltpu.SemaphoreType.DMA,
            pltpu.SemaphoreType.DMA,
            pltpu.SemaphoreType.DMA,
            pltpu.SemaphoreType.DMA,
            pltpu.SemaphoreType.DMA,
            pltpu.SemaphoreType.DMA,
        ],
    )
    def agg(z_hbm, su_u_hbm, sv_u_hbm, su_i_hbm, sv_i_hbm,
            outu_hbm, outi_hbm,
            idx0, idx1, rows0, rows1, val0, val1, gbuf0, gbuf1, obuf,
            srows0, srows1, acc_sh, esem0, esem1, gsem0, gsem1,
            ssem0, ssem1):
        c = lax.axis_index("c")
        s = lax.axis_index("s")
        idxb, rowsb, valb = [idx0, idx1], [rows0, rows1], [val0, val1]
        gb = [gbuf0, gbuf1]
        srows = [srows0, srows1]
        esem, gsem = [esem0, esem1], [gsem0, gsem1]
        ssem = [ssem0, ssem1]
        zero16 = jnp.zeros((LANES,), jnp.float32)

        def run_side(su_hbm, sv_hbm, out_hbm, side_off):
            def e_descs(w, b):
                g = s + NSUB * w
                i = g // NWS
                j = g - i * NWS
                rbase = 2 * i * E + WSZ * j
                vbase = i * E + WSZ * j
                return [
                    pltpu.make_async_copy(su_hbm.at[pl.ds(rbase + E, WSZ)],
                                          idxb[b], esem[b]),
                    pltpu.make_async_copy(su_hbm.at[pl.ds(rbase, WSZ)],
                                          rowsb[b], esem[b]),
                    pltpu.make_async_copy(sv_hbm.at[pl.ds(vbase, WSZ)],
                                          valb[b], esem[b]),
                ]

            def e_start(w, b):
                for cpy in e_descs(w, b):
                    cpy.start()

            def e_wait(w, b):
                for cpy in e_descs(w, b):
                    cpy.wait()
                g = s + NSUB * w
                off = (g // NWS) * (2 * N) + side_off
                voff = jnp.full((LANES,), off, jnp.int32)
                for k in range(WSZ // LANES):
                    sl = pl.ds(k * LANES, LANES)
                    idxb[b][sl] = idxb[b][sl] + voff

            def gcopy(b):
                return pltpu.make_async_copy(z_hbm.at[idxb[b]], gb[b], gsem[b])

            def scopy(b):
                return pltpu.make_async_copy(gb[b], acc_sh.at[srows[b]], ssem[b])

            def valid(w):
                return s + NSUB * w < NW

            e_start(0, 0)
            e_start(1, 1)

            @pl.loop(0, RCH)
            def _(r):
                for k in range(D // LANES):
                    obuf[r, pl.ds(k * LANES, LANES)] = zero16

            @pl.loop(0, 8)
            def _(k):
                ch = s + NSUB * k

                @pl.when(ch < NCH)
                def _():
                    pltpu.sync_copy(obuf, acc_sh.at[pl.ds(ch * RCH, RCH)])

            e_wait(0, 0)
            gcopy(0).start()

            plsc.subcore_barrier()

            def half_iter(w, p):
                q = 1 - p

                @pl.when(w >= 1)
                def _():
                    scopy(q).wait()

                @pl.when(valid(w + 1))
                def _():
                    e_wait(w + 1, q)
                    gcopy(q).start()

                gcopy(p).wait()

                for k in range(WSZ // LANES):
                    sl = pl.ds(k * LANES, LANES)
                    srows[p][sl] = rowsb[p][sl]

                @pl.loop(0, WSZ, unroll=4)
                def _(e):
                    vv = plsc.load_gather(
                        valb[p], [jnp.full((LANES,), e, jnp.int32)])
                    for k in range(D // LANES):
                        sl = pl.ds(k * LANES, LANES)
                        gb[p][e, sl] = gb[p][e, sl] * vv

                pltpu.async_copy(gb[p], acc_sh.at[srows[p]], ssem[p], add=True)

                @pl.when(valid(w + 2))
                def _():
                    e_start(w + 2, p)

            @pl.loop(0, VSLOTS, step=2)
            def _(w0):
                for p in (0, 1):
                    w = w0 + p

                    @pl.when(valid(w))
                    def _():
                        half_iter(w, p)

            @pl.when(s < NW % NSUB)
            def _():
                scopy(0).wait()

            @pl.when(s >= NW % NSUB)
            def _():
                scopy(1).wait()

            plsc.subcore_barrier()

            @pl.loop(0, 8)
            def _(k):
                ch = s + NSUB * k

                @pl.when(ch < NCH)
                def _():
                    pltpu.sync_copy(acc_sh.at[pl.ds(ch * RCH, RCH)], obuf)

                    @pl.loop(0, RCH, unroll=4)
                    def _(r):
                        for kk in range(D // LANES):
                            sl = pl.ds(kk * LANES, LANES)
                            obuf[r, sl] = jnp.maximum(obuf[r, sl], 0.0)

                    pltpu.sync_copy(obuf, out_hbm.at[pl.ds(ch * RCH, RCH)])

        @pl.when(c == 0)
        def _():
            run_side(su_u_hbm, sv_u_hbm, outu_hbm, N)

        @pl.when(c == 1)
        def _():
            run_side(su_i_hbm, sv_i_hbm, outi_hbm, 0)

    return agg(z, su_u, sv_u, su_i, sv_i)


def kernel(user_sup_idx, user_sup_val, item_sup_idx, item_sup_val,
           user_inputs, item_inputs, W):
    x = jnp.concatenate([user_inputs, item_inputs],
                        axis=0).astype(jnp.bfloat16)
    wt = jnp.transpose(W, (2, 0, 1))
    z = _matmul_table(x, wt).reshape(S * 2 * N, D)

    out_u, out_i = _sc_aggregate(z,
                                 user_sup_idx.reshape(-1),
                                 user_sup_val.reshape(-1),
                                 item_sup_idx.reshape(-1),
                                 item_sup_val.reshape(-1))
    return (out_u, out_i)


# speedup vs baseline: 1.0380x; 1.0030x over previous
import dataclasses
import functools

import jax
import jax.numpy as jnp
from jax import lax
from jax.experimental import pallas as pl
from jax.experimental.pallas import tpu as pltpu
from jax.experimental.pallas import tpu_sc as plsc

N = 10000
D = 128
S = 5
E = 120000

NSUB = 16
LANES = 16
WSZ = 96
NWS = E // WSZ
NW = S * NWS
VSLOTS = 392
RCH = 80
NCH = N // RCH


def _matmul_table(x, wt):
    m = x.shape[0]
    mb = 4000
    nb = m // mb

    def body(wt_ref, x_ref, z_ref, wacc_ref):
        i = pl.program_id(1)

        @pl.when(i == 0)
        def _():
            wacc_ref[...] = jnp.zeros_like(wacc_ref)

        wacc_ref[...] += wt_ref[0]
        z_ref[0] = jnp.dot(x_ref[...], wacc_ref[...].astype(jnp.bfloat16),
                           preferred_element_type=jnp.float32)

    return pl.pallas_call(
        body,
        grid=(nb, S),
        in_specs=[
            pl.BlockSpec((1, D, D), lambda j, i: (i, 0, 0)),
            pl.BlockSpec((mb, D), lambda j, i: (j, 0)),
        ],
        out_specs=pl.BlockSpec((1, mb, D), lambda j, i: (i, j, 0)),
        out_shape=jax.ShapeDtypeStruct((S, m, D), jnp.float32),
        scratch_shapes=[pltpu.VMEM((D, D), jnp.float32)],
    )(wt, x)


def _sc_aggregate(z, su_u, sv_u, su_i, sv_i):
    mesh = plsc.VectorSubcoreMesh(core_axis_name="c", subcore_axis_name="s")
    cp = pltpu.CompilerParams()
    if "needs_layout_passes" in pltpu.CompilerParams.__dataclass_fields__:
        cp = dataclasses.replace(cp, needs_layout_passes=False)

    @functools.partial(
        pl.kernel,
        out_type=[jax.ShapeDtypeStruct((N, D), jnp.float32),
                  jax.ShapeDtypeStruct((N, D), jnp.float32)],
        mesh=mesh,
        compiler_params=cp,
        scratch_types=[
            pltpu.VMEM((WSZ,), jnp.int32),
            pltpu.VMEM((WSZ,), jnp.int32),
            pltpu.VMEM((WSZ,), jnp.int32),
            pltpu.VMEM((WSZ,), jnp.int32),
            pltpu.VMEM((WSZ,), jnp.float32),
            pltpu.VMEM((WSZ,), jnp.float32),
            pltpu.VMEM((WSZ, D), jnp.float32),
            pltpu.VMEM((WSZ, D), jnp.float32),
            pltpu.VMEM((RCH, D), jnp.float32),
            pltpu.VMEM((WSZ,), jnp.int32),
            pltpu.VMEM((WSZ,), jnp.int32),
            pltpu.VMEM_SHARED((N, D), jnp.float32),
            pltpu.SemaphoreType.DMA,
            pltpu.SemaphoreType.DMA,
            pltpu.SemaphoreType.DMA,
            pltpu.SemaphoreType.DMA,
            pltpu.SemaphoreType.DMA,
            pltpu.SemaphoreType.DMA,
        ],
    )
    def agg(z_hbm, su_u_hbm, sv_u_hbm, su_i_hbm, sv_i_hbm,
            outu_hbm, outi_hbm,
            idx0, idx1, rows0, rows1, val0, val1, gbuf0, gbuf1, obuf,
            srows0, srows1, acc_sh, esem0, esem1, gsem0, gsem1,
            ssem0, ssem1):
        c = lax.axis_index("c")
        s = lax.axis_index("s")
        idxb, rowsb, valb = [idx0, idx1], [rows0, rows1], [val0, val1]
        gb = [gbuf0, gbuf1]
        srows = [srows0, srows1]
        esem, gsem = [esem0, esem1], [gsem0, gsem1]
        ssem = [ssem0, ssem1]
        zero16 = jnp.zeros((LANES,), jnp.float32)

        def run_side(su_hbm, sv_hbm, out_hbm, side_off):
            def e_descs(w, b):
                g = s + NSUB * w
                i = g // NWS
                j = g - i * NWS
                rbase = 2 * i * E + WSZ * j
                vbase = i * E + WSZ * j
                return [
                    pltpu.make_async_copy(su_hbm.at[pl.ds(rbase + E, WSZ)],
                                          idxb[b], esem[b]),
                    pltpu.make_async_copy(su_hbm.at[pl.ds(rbase, WSZ)],
                                          rowsb[b], esem[b]),
                    pltpu.make_async_copy(sv_hbm.at[pl.ds(vbase, WSZ)],
                                          valb[b], esem[b]),
                ]

            def e_start(w, b):
                for cpy in e_descs(w, b):
                    cpy.start()

            def e_wait(w, b):
                for cpy in e_descs(w, b):
                    cpy.wait()
                g = s + NSUB * w
                off = (g // NWS) * (2 * N) + side_off
                voff = jnp.full((LANES,), off, jnp.int32)
                for k in range(WSZ // LANES):
                    sl = pl.ds(k * LANES, LANES)
                    idxb[b][sl] = idxb[b][sl] + voff

            def gcopy(b):
                return pltpu.make_async_copy(z_hbm.at[idxb[b]], gb[b], gsem[b])

            def scopy(b):
                return pltpu.make_async_copy(gb[b], acc_sh.at[srows[b]], ssem[b])

            def valid(w):
                return s + NSUB * w < NW

            e_start(0, 0)
            e_start(1, 1)

            @pl.loop(0, RCH)
            def _(r):
                for k in range(D // LANES):
                    obuf[r, pl.ds(k * LANES, LANES)] = zero16

            @pl.loop(0, 8)
            def _(k):
                ch = s + NSUB * k

                @pl.when(ch < NCH)
                def _():
                    pltpu.sync_copy(obuf, acc_sh.at[pl.ds(ch * RCH, RCH)])

            e_wait(0, 0)
            gcopy(0).start()

            plsc.subcore_barrier()

            def half_iter(w, p):
                q = 1 - p

                @pl.when(w >= 1)
                def _():
                    scopy(q).wait()

                @pl.when(valid(w + 1))
                def _():
                    e_wait(w + 1, q)
                    gcopy(q).start()

                gcopy(p).wait()

                for k in range(WSZ // LANES):
                    sl = pl.ds(k * LANES, LANES)
                    srows[p][sl] = rowsb[p][sl]

                @pl.loop(0, WSZ, unroll=8)
                def _(e):
                    vv = plsc.load_gather(
                        valb[p], [jnp.full((LANES,), e, jnp.int32)])
                    for k in range(D // LANES):
                        sl = pl.ds(k * LANES, LANES)
                        gb[p][e, sl] = gb[p][e, sl] * vv

                pltpu.async_copy(gb[p], acc_sh.at[srows[p]], ssem[p], add=True)

                @pl.when(valid(w + 2))
                def _():
                    e_start(w + 2, p)

            @pl.loop(0, VSLOTS, step=2)
            def _(w0):
                for p in (0, 1):
                    w = w0 + p

                    @pl.when(valid(w))
                    def _():
                        half_iter(w, p)

            @pl.when(s < NW % NSUB)
            def _():
                scopy(0).wait()

            @pl.when(s >= NW % NSUB)
            def _():
                scopy(1).wait()

            plsc.subcore_barrier()

            @pl.loop(0, 8)
            def _(k):
                ch = s + NSUB * k

                @pl.when(ch < NCH)
                def _():
                    pltpu.sync_copy(acc_sh.at[pl.ds(ch * RCH, RCH)], obuf)

                    @pl.loop(0, RCH, unroll=4)
                    def _(r):
                        for kk in range(D // LANES):
                            sl = pl.ds(kk * LANES, LANES)
                            obuf[r, sl] = jnp.maximum(obuf[r, sl], 0.0)

                    pltpu.sync_copy(obuf, out_hbm.at[pl.ds(ch * RCH, RCH)])

        @pl.when(c == 0)
        def _():
            run_side(su_u_hbm, sv_u_hbm, outu_hbm, N)

        @pl.when(c == 1)
        def _():
            run_side(su_i_hbm, sv_i_hbm, outi_hbm, 0)

    return agg(z, su_u, sv_u, su_i, sv_i)


def kernel(user_sup_idx, user_sup_val, item_sup_idx, item_sup_val,
           user_inputs, item_inputs, W):
    x = jnp.concatenate([user_inputs, item_inputs],
                        axis=0).astype(jnp.bfloat16)
    wt = jnp.transpose(W, (2, 0, 1))
    z = _matmul_table(x, wt).reshape(S * 2 * N, D)

    out_u, out_i = _sc_aggregate(z,
                                 user_sup_idx.reshape(-1),
                                 user_sup_val.reshape(-1),
                                 item_sup_idx.reshape(-1),
                                 item_sup_val.reshape(-1))
    return (out_u, out_i)
